# baseline (speedup 1.0000x reference)
import numpy as np
import jax
import jax.numpy as jnp
from jax import lax
from jax.experimental import pallas as pl
from jax.experimental.pallas import tpu as pltpu
from jax.experimental.pallas import tpu_sc as plsc

HW = 16384
N = 2048
BOUND = np.float32(0.01 * 2048)
WLAST = np.float32(0.01 * 2048 - 21.0)
INV_BOUND = np.float32(1.0 / (0.01 * 2048))
INF = np.float32(np.inf)
L = 16
NW = 32
QPW = HW // NW
NCH = N // L
U = 8
RCAP = 128
SEGS = ((0, 1), (1, 2), (2, 4), (4, 8), (8, 16))


GSTEP = np.float32(2.0 / 127.0)


def _dtm_body(xx_hbm, xy_hbm, out_hbm, px_ref, py_ref, cand_ref, out_ref):
    wid = lax.axis_index("s") * 2 + lax.axis_index("c")
    qbase = wid * QPW
    pltpu.sync_copy(xx_hbm, px_ref)
    pltpu.sync_copy(xy_hbm, py_ref)

    iota = lax.iota(jnp.int32, L)
    inf_v = jnp.full((L,), INF, jnp.float32)

    def merge3(blo, bhi, csort):
        r = jnp.flip(csort)
        l1 = jnp.minimum(bhi, r)
        r2 = jnp.flip(jnp.sort(l1))
        l2 = jnp.minimum(blo, r2)
        h2 = jnp.maximum(blo, r2)
        return jnp.sort(l2), jnp.sort(h2)

    def chunk_append(j, u, qx, qy, t21, ptr_u):
        px = px_ref[pl.ds(j * L, L)]
        py = py_ref[pl.ds(j * L, L)]
        dx = px - qx
        dy = py - qy
        d = dx * dx + dy * dy
        mask = d < t21
        pc = plsc.cumsum(mask.astype(jnp.int32))
        idx = pc + (ptr_u + (RCAP * u - 1))
        plsc.store_scatter(cand_ref, [idx], d, mask=mask)
        return ptr_u + pc[L - 1]

    def drain_all(blo, bhi, ptrs):
        for u in range(U):
            def sub(i, b, u=u):
                c = cand_ref[pl.ds(RCAP * u + i * L, L)]
                c = jnp.where(iota < (ptrs[u] - i * L), c, INF)
                return merge3(b[0], b[1], jnp.sort(c))
            nsub = lax.shift_right_logical(ptrs[u] + (L - 1), 4)
            blo, bhi = lax.fori_loop(0, nsub, sub, (blo, bhi))
        t21 = jnp.full((L,), bhi[4])
        return blo, bhi, t21

    def group_body(g, _):
        def query_body(l, outacc):
            q = qbase + g * L + l
            col = (q & 127).astype(jnp.float32)
            row = lax.shift_right_logical(q, 7).astype(jnp.float32)
            qx = jnp.full((L,), col * GSTEP - 1.0, jnp.float32)
            qy = jnp.full((L,), 1.0 - row * GSTEP, jnp.float32)
            blo, bhi, t21 = inf_v, inf_v, inf_v
            zeros = (jnp.int32(0),) * U
            ptrs = zeros
            for (lo, hi) in SEGS:
                def seg_body(it, p, qx=qx, qy=qy, t21=t21):
                    return tuple(
                        chunk_append(it * U + u, u, qx, qy, t21, p[u])
                        for u in range(U))
                ptrs = lax.fori_loop(lo, hi, seg_body, ptrs)
                blo, bhi, t21 = drain_all(blo, bhi, ptrs)
                ptrs = zeros
            s16 = jnp.sum(blo)
            s5 = jnp.sum(jnp.where(iota < 5, bhi, jnp.float32(0.0)))
            val = (s16 + s5 + bhi[4] * WLAST) * INV_BOUND
            return jnp.where(iota == l, val, outacc)

        outacc = lax.fori_loop(0, L, query_body, inf_v)
        bits = lax.bitcast_convert_type(outacc, jnp.int32)
        y = lax.bitcast_convert_type(
            lax.shift_right_arithmetic(bits, 1) + 0x1FBD1DF5, jnp.float32)
        for _ in range(3):
            y = 0.5 * (y + outacc / y)
        out_ref[pl.ds(g * L, L)] = y
        return 0

    lax.fori_loop(0, QPW // L, group_body, 0)
    pltpu.sync_copy(out_ref, out_hbm.at[pl.ds(qbase, QPW)])


_dtm = pl.kernel(
    _dtm_body,
    out_type=jax.ShapeDtypeStruct((HW,), jnp.float32),
    mesh=plsc.VectorSubcoreMesh(core_axis_name="c", subcore_axis_name="s"),
    compiler_params=pltpu.CompilerParams(needs_layout_passes=False),
    scratch_types=[
        pltpu.VMEM((N,), jnp.float32),
        pltpu.VMEM((N,), jnp.float32),
        pltpu.VMEM((U * RCAP,), jnp.float32),
        pltpu.VMEM((QPW,), jnp.float32),
    ],
)


def kernel(x):
    return _dtm(x[:, 0], x[:, 1]).reshape(128, 128)
